# baseline (speedup 1.0000x reference)
import functools

import jax
import jax.numpy as jnp
from jax import lax
from jax.experimental import pallas as pl
from jax.experimental.pallas import tpu as pltpu
from jax.experimental.pallas import tpu_sc as plsc

VOCAB_SIZE = 25107
EMB_DIM = 100
CTX_LEN = 5
BATCH = 16
NUM_TOKENS = BATCH * CTX_LEN

_SC_INFO = plsc.get_sparse_core_info()
_NC = _SC_INFO.num_cores
_NS = _SC_INFO.num_subcores
ROWS_PER_WORKER = 8
ACTIVE_WORKERS = NUM_TOKENS // ROWS_PER_WORKER


def _sc_gather(idx_flat, emb):
    mesh = plsc.VectorSubcoreMesh(core_axis_name="c", subcore_axis_name="s")

    @functools.partial(
        pl.kernel,
        mesh=mesh,
        out_type=jax.ShapeDtypeStruct((NUM_TOKENS, EMB_DIM), jnp.float32),
        scratch_types=[
            pltpu.VMEM((16,), jnp.int32),
            pltpu.VMEM((ROWS_PER_WORKER, EMB_DIM), jnp.float32),
            pltpu.SemaphoreType.DMA,
        ],
    )
    def gather_kernel(idx_hbm, table_hbm, out_hbm, idx_v, rows_v, sem):
        wid = lax.axis_index("s") * _NC + lax.axis_index("c")

        @pl.when(wid < ACTIVE_WORKERS)
        def _():
            base = wid * ROWS_PER_WORKER
            pltpu.sync_copy(idx_hbm.at[pl.ds(base, ROWS_PER_WORKER)],
                            idx_v.at[pl.ds(0, ROWS_PER_WORKER)])
            idx_vec = idx_v[...]
            copies = []
            for t in range(ROWS_PER_WORKER):
                row = idx_vec[t]
                copies.append(
                    pltpu.async_copy(table_hbm.at[row], rows_v.at[t], sem))
            for c in copies:
                c.wait()
            pltpu.sync_copy(rows_v, out_hbm.at[pl.ds(base, ROWS_PER_WORKER)])

    return gather_kernel(idx_flat, emb)


VOCAB_TILE = 1024
NUM_VOCAB_TILES = pl.cdiv(VOCAB_SIZE, VOCAB_TILE)


def _mlp_kernel(embedded_ref, w1_ref, b1_ref, w2_ref, b2_ref, w3_ref, b3_ref,
                out_ref, h2_ref):
    @pl.when(pl.program_id(0) == 0)
    def _():
        h1 = jnp.maximum(
            jnp.dot(embedded_ref[...], w1_ref[...],
                    preferred_element_type=jnp.float32) + b1_ref[...], 0.0)
        h2_ref[...] = jnp.maximum(
            jnp.dot(h1, w2_ref[...],
                    preferred_element_type=jnp.float32) + b2_ref[...], 0.0)

    out_ref[...] = jnp.dot(h2_ref[...], w3_ref[...],
                           preferred_element_type=jnp.float32) + b3_ref[...]


def kernel(x, emb, W1, b1, W2, b2, W3, b3):
    embedded = _sc_gather(x.reshape(-1).astype(jnp.int32), emb)
    embedded = embedded.reshape(BATCH, CTX_LEN * EMB_DIM)

    out = pl.pallas_call(
        _mlp_kernel,
        grid=(NUM_VOCAB_TILES,),
        in_specs=[
            pl.BlockSpec((BATCH, CTX_LEN * EMB_DIM), lambda i: (0, 0)),
            pl.BlockSpec((CTX_LEN * EMB_DIM, 300), lambda i: (0, 0)),
            pl.BlockSpec((1, 300), lambda i: (0, 0)),
            pl.BlockSpec((300, 300), lambda i: (0, 0)),
            pl.BlockSpec((1, 300), lambda i: (0, 0)),
            pl.BlockSpec((300, VOCAB_TILE), lambda i: (0, i)),
            pl.BlockSpec((1, VOCAB_TILE), lambda i: (0, i)),
        ],
        out_specs=pl.BlockSpec((BATCH, VOCAB_TILE), lambda i: (0, i)),
        out_shape=jax.ShapeDtypeStruct((BATCH, VOCAB_SIZE), jnp.float32),
        scratch_shapes=[pltpu.VMEM((BATCH, 300), jnp.float32)],
    )(embedded, W1, b1.reshape(1, -1), W2, b2.reshape(1, -1), W3,
      b3.reshape(1, -1))
    return out


# speedup vs baseline: 1.1300x; 1.1300x over previous
import functools

import jax
import jax.numpy as jnp
from jax import lax
from jax.experimental import pallas as pl
from jax.experimental.pallas import tpu as pltpu
from jax.experimental.pallas import tpu_sc as plsc

VOCAB_SIZE = 25107
EMB_DIM = 100
CTX_LEN = 5
BATCH = 16
NUM_TOKENS = BATCH * CTX_LEN

_SC_INFO = plsc.get_sparse_core_info()
_NC = _SC_INFO.num_cores
_NS = _SC_INFO.num_subcores
ROWS_PER_WORKER = 8
ACTIVE_WORKERS = NUM_TOKENS // ROWS_PER_WORKER


def _sc_gather(idx_flat, emb):
    mesh = plsc.VectorSubcoreMesh(core_axis_name="c", subcore_axis_name="s")

    @functools.partial(
        pl.kernel,
        mesh=mesh,
        out_type=jax.ShapeDtypeStruct((NUM_TOKENS, EMB_DIM), jnp.float32),
        scratch_types=[
            pltpu.VMEM((16,), jnp.int32),
            pltpu.VMEM((ROWS_PER_WORKER, EMB_DIM), jnp.float32),
            pltpu.SemaphoreType.DMA,
        ],
    )
    def gather_kernel(idx_hbm, table_hbm, out_hbm, idx_v, rows_v, sem):
        wid = lax.axis_index("s") * _NC + lax.axis_index("c")

        @pl.when(wid < ACTIVE_WORKERS)
        def _():
            base = wid * ROWS_PER_WORKER
            pltpu.sync_copy(idx_hbm.at[pl.ds(base, ROWS_PER_WORKER)],
                            idx_v.at[pl.ds(0, ROWS_PER_WORKER)])
            idx_vec = idx_v[...]
            copies = []
            for t in range(ROWS_PER_WORKER):
                row = idx_vec[t]
                copies.append(
                    pltpu.async_copy(table_hbm.at[row], rows_v.at[t], sem))
            for c in copies:
                c.wait()
            pltpu.sync_copy(rows_v, out_hbm.at[pl.ds(base, ROWS_PER_WORKER)])

    return gather_kernel(idx_flat, emb)


VOCAB_TILE = 2048
NUM_VOCAB_TILES = pl.cdiv(VOCAB_SIZE, VOCAB_TILE)


def _mlp_kernel(embedded_ref, w1_ref, b1_ref, w2_ref, b2_ref, w3_ref, b3_ref,
                out_ref, h2_ref):
    @pl.when(pl.program_id(0) == 0)
    def _():
        h1 = jnp.maximum(
            jnp.dot(embedded_ref[...], w1_ref[...],
                    preferred_element_type=jnp.float32) + b1_ref[...], 0.0)
        h2_ref[...] = jnp.maximum(
            jnp.dot(h1, w2_ref[...],
                    preferred_element_type=jnp.float32) + b2_ref[...], 0.0)

    out_ref[...] = jnp.dot(h2_ref[...], w3_ref[...],
                           preferred_element_type=jnp.float32) + b3_ref[...]


def kernel(x, emb, W1, b1, W2, b2, W3, b3):
    embedded = _sc_gather(x.reshape(-1).astype(jnp.int32), emb)
    embedded = embedded.reshape(BATCH, CTX_LEN * EMB_DIM)

    out = pl.pallas_call(
        _mlp_kernel,
        grid=(NUM_VOCAB_TILES,),
        in_specs=[
            pl.BlockSpec((BATCH, CTX_LEN * EMB_DIM), lambda i: (0, 0)),
            pl.BlockSpec((CTX_LEN * EMB_DIM, 300), lambda i: (0, 0)),
            pl.BlockSpec((1, 300), lambda i: (0, 0)),
            pl.BlockSpec((300, 300), lambda i: (0, 0)),
            pl.BlockSpec((1, 300), lambda i: (0, 0)),
            pl.BlockSpec((300, VOCAB_TILE), lambda i: (0, i)),
            pl.BlockSpec((1, VOCAB_TILE), lambda i: (0, i)),
        ],
        out_specs=pl.BlockSpec((BATCH, VOCAB_TILE), lambda i: (0, i)),
        out_shape=jax.ShapeDtypeStruct((BATCH, VOCAB_SIZE), jnp.float32),
        scratch_shapes=[pltpu.VMEM((BATCH, 300), jnp.float32)],
    )(embedded, W1, b1.reshape(1, -1), W2, b2.reshape(1, -1), W3,
      b3.reshape(1, -1))
    return out


# speedup vs baseline: 1.2030x; 1.0646x over previous
import functools

import jax
import jax.numpy as jnp
from jax import lax
from jax.experimental import pallas as pl
from jax.experimental.pallas import tpu as pltpu
from jax.experimental.pallas import tpu_sc as plsc

VOCAB_SIZE = 25107
EMB_DIM = 100
CTX_LEN = 5
BATCH = 16
NUM_TOKENS = BATCH * CTX_LEN

_SC_INFO = plsc.get_sparse_core_info()
_NC = _SC_INFO.num_cores
_NS = _SC_INFO.num_subcores
ROWS_PER_WORKER = 8
ACTIVE_WORKERS = NUM_TOKENS // ROWS_PER_WORKER


def _sc_gather(idx_flat, emb):
    mesh = plsc.VectorSubcoreMesh(core_axis_name="c", subcore_axis_name="s")

    @functools.partial(
        pl.kernel,
        mesh=mesh,
        out_type=jax.ShapeDtypeStruct((NUM_TOKENS, EMB_DIM), jnp.float32),
        scratch_types=[
            pltpu.VMEM((16,), jnp.int32),
            pltpu.VMEM((ROWS_PER_WORKER, EMB_DIM), jnp.float32),
            pltpu.SemaphoreType.DMA,
        ],
    )
    def gather_kernel(idx_hbm, table_hbm, out_hbm, idx_v, rows_v, sem):
        wid = lax.axis_index("s") * _NC + lax.axis_index("c")

        @pl.when(wid < ACTIVE_WORKERS)
        def _():
            base = wid * ROWS_PER_WORKER
            pltpu.sync_copy(idx_hbm.at[pl.ds(base, ROWS_PER_WORKER)],
                            idx_v.at[pl.ds(0, ROWS_PER_WORKER)])
            idx_vec = idx_v[...]
            copies = []
            for t in range(ROWS_PER_WORKER):
                row = idx_vec[t]
                copies.append(
                    pltpu.async_copy(table_hbm.at[row], rows_v.at[t], sem))
            for c in copies:
                c.wait()
            pltpu.sync_copy(rows_v, out_hbm.at[pl.ds(base, ROWS_PER_WORKER)])

    return gather_kernel(idx_flat, emb)


VOCAB_TILE = 4096
NUM_VOCAB_TILES = pl.cdiv(VOCAB_SIZE, VOCAB_TILE)


def _mlp_kernel(embedded_ref, w1_ref, b1_ref, w2_ref, b2_ref, w3_ref, b3_ref,
                out_ref, h2_ref):
    @pl.when(pl.program_id(0) == 0)
    def _():
        h1 = jnp.maximum(
            jnp.dot(embedded_ref[...], w1_ref[...],
                    preferred_element_type=jnp.float32) + b1_ref[...], 0.0)
        h2_ref[...] = jnp.maximum(
            jnp.dot(h1, w2_ref[...],
                    preferred_element_type=jnp.float32) + b2_ref[...], 0.0)

    out_ref[...] = jnp.dot(h2_ref[...], w3_ref[...],
                           preferred_element_type=jnp.float32) + b3_ref[...]


def kernel(x, emb, W1, b1, W2, b2, W3, b3):
    embedded = _sc_gather(x.reshape(-1).astype(jnp.int32), emb)
    embedded = embedded.reshape(BATCH, CTX_LEN * EMB_DIM)

    out = pl.pallas_call(
        _mlp_kernel,
        grid=(NUM_VOCAB_TILES,),
        in_specs=[
            pl.BlockSpec((BATCH, CTX_LEN * EMB_DIM), lambda i: (0, 0)),
            pl.BlockSpec((CTX_LEN * EMB_DIM, 300), lambda i: (0, 0)),
            pl.BlockSpec((1, 300), lambda i: (0, 0)),
            pl.BlockSpec((300, 300), lambda i: (0, 0)),
            pl.BlockSpec((1, 300), lambda i: (0, 0)),
            pl.BlockSpec((300, VOCAB_TILE), lambda i: (0, i)),
            pl.BlockSpec((1, VOCAB_TILE), lambda i: (0, i)),
        ],
        out_specs=pl.BlockSpec((BATCH, VOCAB_TILE), lambda i: (0, i)),
        out_shape=jax.ShapeDtypeStruct((BATCH, VOCAB_SIZE), jnp.float32),
        scratch_shapes=[pltpu.VMEM((BATCH, 300), jnp.float32)],
    )(embedded, W1, b1.reshape(1, -1), W2, b2.reshape(1, -1), W3,
      b3.reshape(1, -1))
    return out
